# Initial kernel scaffold; baseline (speedup 1.0000x reference)
import functools

import jax
import jax.numpy as jnp
from jax import lax
from jax.experimental import pallas as pl
from jax.experimental.pallas import tpu as pltpu
from jax.experimental.pallas import tpu_sc as plsc

EMBED_DIM = 64
CHUNK = 512


@functools.cache
def _make_kernel(n_idx: int):
    info = plsc.get_sparse_core_info()
    num_cores = info.num_cores
    num_workers = info.num_cores * info.num_subcores
    b_per_w = n_idx // num_workers
    assert n_idx % num_workers == 0 and b_per_w % CHUNK == 0
    n_chunks = b_per_w // CHUNK

    mesh = plsc.VectorSubcoreMesh(core_axis_name="c", subcore_axis_name="s")

    @functools.partial(
        pl.kernel,
        mesh=mesh,
        out_type=jax.ShapeDtypeStruct((n_idx, EMBED_DIM), jnp.float32),
        scratch_types=[
            pltpu.VMEM((CHUNK,), jnp.int32),
            pltpu.VMEM((CHUNK, EMBED_DIM), jnp.float32),
            pltpu.SemaphoreType.DMA,
        ],
    )
    def emb_kernel(idx_hbm, table_hbm, out_hbm, idx_v, rows_v, sem):
        wid = lax.axis_index("s") * num_cores + lax.axis_index("c")
        base = wid * b_per_w

        def body(i, carry):
            off = base + i * CHUNK
            pltpu.sync_copy(idx_hbm.at[pl.ds(off, CHUNK)], idx_v)
            pltpu.async_copy(table_hbm.at[idx_v], rows_v, sem).wait()
            pltpu.sync_copy(rows_v, out_hbm.at[pl.ds(off, CHUNK)])
            return carry

        lax.fori_loop(0, n_chunks, body, 0)

    return emb_kernel


@jax.jit
def kernel(input_id, table):
    batch, seq_len = input_id.shape
    flat_idx = input_id.reshape(batch * seq_len)
    out = _make_kernel(batch * seq_len)(flat_idx, table)
    return out.reshape(batch, seq_len, EMBED_DIM)


# speedup vs baseline: 3.9473x; 3.9473x over previous
import functools

import jax
import jax.numpy as jnp
from jax import lax
from jax.experimental import pallas as pl
from jax.experimental.pallas import tpu as pltpu
from jax.experimental.pallas import tpu_sc as plsc

EMBED_DIM = 64
CHUNK = 512


@functools.cache
def _make_kernel(n_idx: int):
    info = plsc.get_sparse_core_info()
    num_cores = info.num_cores
    num_workers = info.num_cores * info.num_subcores
    b_per_w = n_idx // num_workers
    assert n_idx % num_workers == 0 and b_per_w % CHUNK == 0
    n_chunks = b_per_w // CHUNK

    mesh = plsc.VectorSubcoreMesh(core_axis_name="c", subcore_axis_name="s")

    @functools.partial(
        pl.kernel,
        mesh=mesh,
        out_type=jax.ShapeDtypeStruct((n_idx, EMBED_DIM), jnp.float32),
        scratch_types=[
            pltpu.VMEM((CHUNK,), jnp.int32),
            pltpu.VMEM((CHUNK, EMBED_DIM), jnp.float32),
            pltpu.SemaphoreType.DMA,
        ],
        compiler_params=pltpu.CompilerParams(use_tc_tiling_on_sc=False),
    )
    def emb_kernel(idx_hbm, table_hbm, out_hbm, idx_v, rows_v, sem):
        wid = lax.axis_index("s") * num_cores + lax.axis_index("c")
        base = wid * b_per_w

        def body(i, carry):
            off = base + i * CHUNK
            pltpu.sync_copy(idx_hbm.at[pl.ds(off, CHUNK)], idx_v)
            pltpu.async_copy(table_hbm.at[idx_v], rows_v, sem).wait()
            pltpu.sync_copy(rows_v, out_hbm.at[pl.ds(off, CHUNK)])
            return carry

        lax.fori_loop(0, n_chunks, body, 0)

    return emb_kernel


@jax.jit
def kernel(input_id, table):
    batch, seq_len = input_id.shape
    flat_idx = input_id.reshape(batch * seq_len)
    out = _make_kernel(batch * seq_len)(flat_idx, table)
    return out.reshape(batch, seq_len, EMBED_DIM)


# speedup vs baseline: 5.5792x; 1.4134x over previous
import functools

import jax
import jax.numpy as jnp
from jax import lax
from jax.experimental import pallas as pl
from jax.experimental.pallas import tpu as pltpu
from jax.experimental.pallas import tpu_sc as plsc

EMBED_DIM = 64
PAD_DIM = 128
CHUNK = 256
NBUF = 2


@functools.cache
def _make_kernel(n_idx: int):
    info = plsc.get_sparse_core_info()
    num_cores = info.num_cores
    num_workers = info.num_cores * info.num_subcores
    b_per_w = n_idx // num_workers
    assert n_idx % num_workers == 0 and b_per_w % CHUNK == 0
    n_chunks = b_per_w // CHUNK
    n_groups = n_chunks // NBUF
    assert n_chunks % NBUF == 0 and n_groups >= 3

    mesh = plsc.VectorSubcoreMesh(core_axis_name="c", subcore_axis_name="s")

    @functools.partial(
        pl.kernel,
        mesh=mesh,
        out_type=jax.ShapeDtypeStruct((n_idx, PAD_DIM), jnp.float32),
        scratch_types=[
            pltpu.VMEM((b_per_w,), jnp.int32),
            pltpu.VMEM((NBUF, CHUNK, PAD_DIM), jnp.float32),
            pltpu.SemaphoreType.DMA,
            pltpu.SemaphoreType.DMA,
        ],
    )
    def emb_kernel(idx_hbm, table_hbm, out_hbm, idx_v, rows_v, gsem, osem):
        wid = lax.axis_index("s") * num_cores + lax.axis_index("c")
        base = wid * b_per_w
        pltpu.sync_copy(idx_hbm.at[pl.ds(base, b_per_w)], idx_v)

        def start_gather(i, b):
            off = pl.multiple_of(i * CHUNK, CHUNK)
            pltpu.async_copy(
                table_hbm.at[idx_v.at[pl.ds(off, CHUNK)]], rows_v.at[b], gsem)

        def wait_gather(b):
            pltpu.make_async_copy(
                table_hbm.at[idx_v.at[pl.ds(0, CHUNK)]], rows_v.at[b], gsem
            ).wait()

        def start_out(i, b):
            off = pl.multiple_of(base + i * CHUNK, CHUNK)
            pltpu.async_copy(rows_v.at[b], out_hbm.at[pl.ds(off, CHUNK)], osem)

        def wait_out(b):
            pltpu.make_async_copy(
                rows_v.at[b], out_hbm.at[pl.ds(0, CHUNK)], osem
            ).wait()

        def visit(i, b, retire_prev=True, start_next=True):
            wait_gather(b)
            start_out(i, b)
            if retire_prev:
                wait_out((b - 1) % NBUF)
            if start_next:
                start_gather(i - 1 + NBUF, (b - 1) % NBUF)

        for b in range(NBUF):
            start_gather(b, b)

        for b in range(NBUF):
            visit(b, b, retire_prev=b >= 1, start_next=b >= 1)

        @pl.loop(1, n_groups - 1)
        def _(t):
            for b in range(NBUF):
                visit(t * NBUF + b, b, start_next=True)

        for b in range(NBUF):
            i = (n_groups - 1) * NBUF + b
            visit(i, b, start_next=(i - 1 + NBUF) < n_chunks)

        wait_out((n_chunks - 1) % NBUF)

    return emb_kernel


@jax.jit
def kernel(input_id, table):
    batch, seq_len = input_id.shape
    flat_idx = input_id.reshape(batch * seq_len)
    padded = jnp.pad(table, ((0, 0), (0, PAD_DIM - EMBED_DIM)))
    out = _make_kernel(batch * seq_len)(flat_idx, padded)
    return out[:, :EMBED_DIM].reshape(batch, seq_len, EMBED_DIM)


# speedup vs baseline: 5.5965x; 1.0031x over previous
import functools

import jax
import jax.numpy as jnp
from jax import lax
from jax.experimental import pallas as pl
from jax.experimental.pallas import tpu as pltpu
from jax.experimental.pallas import tpu_sc as plsc

EMBED_DIM = 64
PAD_DIM = 128
CHUNK = 128
NBUF = 4


@functools.cache
def _make_kernel(n_idx: int):
    info = plsc.get_sparse_core_info()
    num_cores = info.num_cores
    num_workers = info.num_cores * info.num_subcores
    b_per_w = n_idx // num_workers
    assert n_idx % num_workers == 0 and b_per_w % CHUNK == 0
    n_chunks = b_per_w // CHUNK
    n_groups = n_chunks // NBUF
    assert n_chunks % NBUF == 0 and n_groups >= 3

    mesh = plsc.VectorSubcoreMesh(core_axis_name="c", subcore_axis_name="s")

    @functools.partial(
        pl.kernel,
        mesh=mesh,
        out_type=jax.ShapeDtypeStruct((n_idx, PAD_DIM), jnp.float32),
        scratch_types=[
            pltpu.VMEM((b_per_w,), jnp.int32),
            pltpu.VMEM((NBUF, CHUNK, PAD_DIM), jnp.float32),
            pltpu.SemaphoreType.DMA,
            pltpu.SemaphoreType.DMA,
        ],
    )
    def emb_kernel(idx_hbm, table_hbm, out_hbm, idx_v, rows_v, gsem, osem):
        wid = lax.axis_index("s") * num_cores + lax.axis_index("c")
        base = wid * b_per_w
        pltpu.sync_copy(idx_hbm.at[pl.ds(base, b_per_w)], idx_v)

        def start_gather(i, b):
            off = pl.multiple_of(i * CHUNK, CHUNK)
            pltpu.async_copy(
                table_hbm.at[idx_v.at[pl.ds(off, CHUNK)]], rows_v.at[b], gsem)

        def wait_gather(b):
            pltpu.make_async_copy(
                table_hbm.at[idx_v.at[pl.ds(0, CHUNK)]], rows_v.at[b], gsem
            ).wait()

        def start_out(i, b):
            off = pl.multiple_of(base + i * CHUNK, CHUNK)
            pltpu.async_copy(rows_v.at[b], out_hbm.at[pl.ds(off, CHUNK)], osem)

        def wait_out(b):
            pltpu.make_async_copy(
                rows_v.at[b], out_hbm.at[pl.ds(0, CHUNK)], osem
            ).wait()

        def visit(i, b, retire_prev=True, start_next=True):
            wait_gather(b)
            start_out(i, b)
            if retire_prev:
                wait_out((b - 1) % NBUF)
            if start_next:
                start_gather(i - 1 + NBUF, (b - 1) % NBUF)

        for b in range(NBUF):
            start_gather(b, b)

        for b in range(NBUF):
            visit(b, b, retire_prev=b >= 1, start_next=b >= 1)

        @pl.loop(1, n_groups - 1)
        def _(t):
            for b in range(NBUF):
                visit(t * NBUF + b, b, start_next=True)

        for b in range(NBUF):
            i = (n_groups - 1) * NBUF + b
            visit(i, b, start_next=(i - 1 + NBUF) < n_chunks)

        wait_out((n_chunks - 1) % NBUF)

    return emb_kernel


@jax.jit
def kernel(input_id, table):
    batch, seq_len = input_id.shape
    flat_idx = input_id.reshape(batch * seq_len)
    padded = jnp.pad(table, ((0, 0), (0, PAD_DIM - EMBED_DIM)))
    out = _make_kernel(batch * seq_len)(flat_idx, padded)
    return out[:, :EMBED_DIM].reshape(batch, seq_len, EMBED_DIM)
